# baseline (speedup 1.0000x reference)
import functools

import jax
import jax.numpy as jnp
from jax import lax
from jax.experimental import pallas as pl
from jax.experimental.pallas import tpu as pltpu
from jax.experimental.pallas import tpu_sc as plsc

BATCH = 4096
DIM = 64
NUM_TABLES = 6

_info = plsc.get_sparse_core_info()
_NC, _NS = _info.num_cores, _info.num_subcores
_NW = _NC * _NS
_BPW = BATCH // _NW
_CR = 4
_GROUP = 16


def _make():
  mesh = plsc.VectorSubcoreMesh(core_axis_name="c", subcore_axis_name="s")

  @functools.partial(
      pl.kernel,
      mesh=mesh,
      out_type=jax.ShapeDtypeStruct((BATCH, NUM_TABLES * DIM), jnp.float32),
      compiler_params=pltpu.CompilerParams(use_tc_tiling_on_sc=True),
      scratch_types=[
          pltpu.VMEM((_BPW,), jnp.int32),
          pltpu.VMEM((_BPW,), jnp.int32),
          pltpu.VMEM((2, NUM_TABLES, _CR, 8, DIM), jnp.float32),
          pltpu.VMEM((NUM_TABLES // 2, _BPW, 2 * DIM), jnp.float32),
          pltpu.SemaphoreType.DMA,
          pltpu.SemaphoreType.DMA,
          pltpu.SemaphoreType.DMA,
      ],
  )
  def lookup(u_hbm, i_hbm, t0, t1, t2, t3, t4, t5, out_hbm,
             uid_v, iid_v, pairs, gbuf, sem0, sem1, wsem):
    wid = lax.axis_index("s") * _NC + lax.axis_index("c")
    base = wid * _BPW
    pltpu.sync_copy(u_hbm.at[pl.ds(base, _BPW)], uid_v)
    pltpu.sync_copy(i_hbm.at[pl.ds(base, _BPW)], iid_v)
    tables = (t0, t1, t2, t3, t4, t5)
    sems = (sem0, sem1)

    def fire(ue, ie, g2, buf, sem):
      for j in range(_CR):
        jj = g2 * _CR + j
        for k in range(NUM_TABLES):
          ev = pl.multiple_of((ue if k % 2 == 0 else ie)[jj], 8)
          pltpu.async_copy(tables[k].at[pl.ds(ev, 8), :],
                           pairs.at[buf, k, j], sem)

    def drain(buf, sem):
      for j in range(_CR):
        for k in range(NUM_TABLES):
          pltpu.make_async_copy(tables[0].at[pl.ds(0, 8), :],
                                pairs.at[buf, k, j], sem).wait()

    def repack(uh, ih, gg, g2, buf):
      for j in range(_CR):
        jj = g2 * _CR + j
        rr = gg * _GROUP + jj
        for k in range(NUM_TABLES):
          h = (uh if k % 2 == 0 else ih)[jj]
          for q in range(DIM // 16):
            v = pairs[buf, k, j, h, pl.ds(q * 16, 16)]
            gbuf[k // 2, rr, pl.ds((k % 2) * DIM + q * 16, 16)] = v

    def group(gg, carry):
      uv = uid_v[pl.ds(gg * _GROUP, _GROUP)]
      iv = iid_v[pl.ds(gg * _GROUP, _GROUP)]
      ue = uv & jnp.int32(~7)
      ie = iv & jnp.int32(~7)
      uh = uv & 7
      ih = iv & 7
      fire(ue, ie, 0, 0, sems[0])
      fire(ue, ie, 1, 1, sems[1])
      drain(0, sems[0])
      repack(uh, ih, gg, 0, 0)
      fire(ue, ie, 2, 0, sems[0])
      drain(1, sems[1])
      repack(uh, ih, gg, 1, 1)
      fire(ue, ie, 3, 1, sems[1])
      drain(0, sems[0])
      repack(uh, ih, gg, 2, 0)
      drain(1, sems[1])
      repack(uh, ih, gg, 3, 1)
      return carry

    lax.fori_loop(0, _BPW // _GROUP, group, 0)

    for p in range(NUM_TABLES // 2):
      pltpu.async_copy(
          gbuf.at[p],
          out_hbm.at[pl.ds(base, _BPW), pl.ds(p * 2 * DIM, 2 * DIM)], wsem)
    for p in range(NUM_TABLES // 2):
      pltpu.make_async_copy(
          gbuf.at[p],
          out_hbm.at[pl.ds(base, _BPW), pl.ds(p * 2 * DIM, 2 * DIM)],
          wsem).wait()

  return lookup


_lookup = _make()


def kernel(uid, iid, user_table, item_table, src_user_0, src_item_0,
           src_user_1, src_item_1):
  return _lookup(uid.astype(jnp.int32), iid.astype(jnp.int32),
                 user_table, item_table, src_user_0, src_item_0,
                 src_user_1, src_item_1)


# speedup vs baseline: 1.1917x; 1.1917x over previous
import functools

import jax
import jax.numpy as jnp
from jax import lax
from jax.experimental import pallas as pl
from jax.experimental.pallas import tpu as pltpu
from jax.experimental.pallas import tpu_sc as plsc

BATCH = 4096
DIM = 64
NUM_TABLES = 6

_info = plsc.get_sparse_core_info()
_NC, _NS = _info.num_cores, _info.num_subcores
_NW = _NC * _NS
_BPW = BATCH // _NW


def _make():
  mesh = plsc.VectorSubcoreMesh(core_axis_name="c", subcore_axis_name="s")

  @functools.partial(
      pl.kernel,
      mesh=mesh,
      out_type=jax.ShapeDtypeStruct((BATCH, NUM_TABLES * DIM), jnp.float32),
      compiler_params=pltpu.CompilerParams(
          use_tc_tiling_on_sc=True,
          disable_bounds_checks=True,
          disable_semaphore_checks=True,
      ),
      scratch_types=[
          pltpu.VMEM((_BPW,), jnp.int32),
          pltpu.VMEM((_BPW,), jnp.int32),
          pltpu.VMEM((NUM_TABLES // 2, _BPW, 2 * DIM), jnp.float32),
          pltpu.SemaphoreType.DMA,
          pltpu.SemaphoreType.DMA,
          pltpu.SemaphoreType.DMA,
          pltpu.SemaphoreType.DMA,
      ],
  )
  def lookup(u_hbm, i_hbm, t0, t1, t2, t3, t4, t5, out_hbm,
             uid_v, iid_v, gbuf, s0, s1, s2, wsem):
    wid = lax.axis_index("s") * _NC + lax.axis_index("c")
    base = wid * _BPW
    pltpu.sync_copy(u_hbm.at[pl.ds(base, _BPW)], uid_v)
    pltpu.sync_copy(i_hbm.at[pl.ds(base, _BPW)], iid_v)
    tables = (t0, t1, t2, t3, t4, t5)
    sems = (s0, s1, s2)

    def fire(p):
      def body(c, carry):
        uv = uid_v[pl.ds(c * 16, 16)]
        iv = iid_v[pl.ds(c * 16, 16)]
        for j in range(16):
          r = c * 16 + j
          for k in (2 * p, 2 * p + 1):
            idx = uv[j] if k % 2 == 0 else iv[j]
            pltpu.async_copy(
                tables[k].at[idx],
                gbuf.at[p, r, pl.ds((k % 2) * DIM, DIM)], sems[p])
        return carry

      lax.fori_loop(0, _BPW // 16, body, 0)

    for p in range(NUM_TABLES // 2):
      fire(p)

    for p in range(NUM_TABLES // 2):
      pltpu.make_async_copy(
          out_hbm.at[pl.ds(base, _BPW), pl.ds(p * 2 * DIM, 2 * DIM)],
          gbuf.at[p], sems[p]).wait()
      pltpu.async_copy(
          gbuf.at[p],
          out_hbm.at[pl.ds(base, _BPW), pl.ds(p * 2 * DIM, 2 * DIM)], wsem)
    for p in range(NUM_TABLES // 2):
      pltpu.make_async_copy(
          gbuf.at[p],
          out_hbm.at[pl.ds(base, _BPW), pl.ds(p * 2 * DIM, 2 * DIM)],
          wsem).wait()

  return lookup


_lookup = _make()


def kernel(uid, iid, user_table, item_table, src_user_0, src_item_0,
           src_user_1, src_item_1):
  return _lookup(uid.astype(jnp.int32), iid.astype(jnp.int32),
                 user_table, item_table, src_user_0, src_item_0,
                 src_user_1, src_item_1)
